# baseline (speedup 1.0000x reference)
import functools

import jax
import jax.numpy as jnp
from jax import lax
from jax.experimental import pallas as pl
from jax.experimental.pallas import tpu as pltpu
from jax.experimental.pallas import tpu_sc as plsc

_B, _P, _L, _M = 4, 8, 1024, 2048
_N = _B * _P
_LAN = 16
_KL = 8

_mesh = plsc.VectorSubcoreMesh(
    core_axis_name="c", subcore_axis_name="s", num_cores=2, num_subcores=16
)


@functools.partial(
    pl.kernel,
    out_type=jax.ShapeDtypeStruct((_N, 2 * _LAN), jnp.float32),
    mesh=_mesh,
    scratch_types=[
        pltpu.VMEM((_L + _LAN,), jnp.float32),
        pltpu.VMEM((_L + _LAN,), jnp.float32),
        pltpu.VMEM((_M,), jnp.float32),
        pltpu.VMEM((_M,), jnp.float32),
        pltpu.VMEM((_M,), jnp.float32),
        pltpu.VMEM((_N,), jnp.int32),
        pltpu.VMEM((_N,), jnp.int32),
        pltpu.VMEM((2 * _LAN,), jnp.float32),
    ],
    compiler_params=pltpu.CompilerParams(needs_layout_passes=False),
)
def _chamfer_sc(bx, by, ex, ey, xl, yl, out, x0s, x1s, y0s, y1s, mny, xls, yls, ost):
    mid = lax.axis_index("s") * 2 + lax.axis_index("c")
    pltpu.sync_copy(bx.at[mid], x0s.at[pl.ds(0, _L)])
    pltpu.sync_copy(by.at[mid], x1s.at[pl.ds(0, _L)])
    pltpu.sync_copy(ex.at[mid], y0s)
    pltpu.sync_copy(ey.at[mid], y1s)
    pltpu.sync_copy(xl, xls)
    pltpu.sync_copy(yl, yls)
    iot = lax.iota(jnp.int32, _LAN)
    c16 = (mid // _LAN) * _LAN
    sel = (c16 + iot) == mid
    nx = jnp.max(jnp.where(sel, xls[pl.ds(c16, _LAN)], 0))
    ny = jnp.max(jnp.where(sel, yls[pl.ds(c16, _LAN)], 0))

    wb = jnp.minimum((nx // _LAN) * _LAN, _L - _LAN)
    mx = (wb + iot) < nx
    x0s[pl.ds(wb, _LAN)] = jnp.where(mx, x0s[pl.ds(wb, _LAN)], 1.0e4)
    x1s[pl.ds(wb, _LAN)] = jnp.where(mx, x1s[pl.ds(wb, _LAN)], 1.0e4)
    vb = jnp.minimum((ny // _LAN) * _LAN, _M - _LAN)
    my = (vb + iot) < ny
    y0s[pl.ds(vb, _LAN)] = jnp.where(my, y0s[pl.ds(vb, _LAN)], 2.0e4)
    y1s[pl.ds(vb, _LAN)] = jnp.where(my, y1s[pl.ds(vb, _LAN)], 2.0e4)

    ncy = (ny + _LAN - 1) // _LAN
    nbl = (nx + _KL - 1) // _KL
    big = jnp.full((_LAN,), 1.0e10, jnp.float32)

    def init_b(mc, c):
        mny[pl.ds(mc * _LAN, _LAN)] = big
        return c

    lax.fori_loop(0, ncy, init_b, 0)

    def lblk(blk, cham_x):
        lb = blk * _KL
        xv0 = x0s[pl.ds(lb, _LAN)]
        xv1 = x1s[pl.ds(lb, _LAN)]
        xb0 = [jnp.full((_LAN,), xv0[i]) for i in range(_KL)]
        xb1 = [jnp.full((_LAN,), xv1[i]) for i in range(_KL)]

        def mstep(mc, accs):
            mb = mc * _LAN
            v0 = y0s[pl.ds(mb, _LAN)]
            v1 = y1s[pl.ds(mb, _LAN)]
            mn = mny[pl.ds(mb, _LAN)]
            nacc = []
            for i in range(_KL):
                dx = xb0[i] - v0
                dy = xb1[i] - v1
                d2 = dx * dx + dy * dy
                nacc.append(jnp.minimum(accs[i], d2))
                mn = jnp.minimum(mn, d2)
            mny[pl.ds(mb, _LAN)] = mn
            return tuple(nacc)

        accs = lax.fori_loop(0, ncy, mstep, (big,) * _KL)
        for i in range(_KL):
            rm = jnp.min(accs[i])
            cham_x = cham_x + jnp.where(lb + i < nx, rm, jnp.float32(0.0))
        return cham_x

    cham_x = lax.fori_loop(0, nbl, lblk, jnp.asarray(0.0, jnp.float32))

    def sum_b(mc, sv):
        mb = mc * _LAN
        valid = (mb + iot) < ny
        return sv + jnp.where(valid, mny[pl.ds(mb, _LAN)], jnp.float32(0.0))

    syv = lax.fori_loop(0, ncy, sum_b, jnp.zeros((_LAN,), jnp.float32))

    ost[pl.ds(0, _LAN)] = jnp.full((_LAN,), cham_x)
    ost[pl.ds(_LAN, _LAN)] = syv
    pltpu.sync_copy(ost, out.at[mid])


def kernel(boundaries, edgemaps, boundary_lengths, edgemaps_len):
    bx = boundaries[..., 0].reshape(_N, _L)
    by = boundaries[..., 1].reshape(_N, _L)
    ex = edgemaps[..., 0].reshape(_N, _M)
    ey = edgemaps[..., 1].reshape(_N, _M)
    xl = boundary_lengths.reshape(_N).astype(jnp.int32)
    yl = edgemaps_len.reshape(_N).astype(jnp.int32)
    o = _chamfer_sc(bx, by, ex, ey, xl, yl)
    xlf = xl.astype(jnp.float32)
    ylf = yl.astype(jnp.float32)
    loss = o[:, 0] / xlf + o[:, _LAN:].sum(axis=1) / ylf
    return loss.reshape(_B, _P).mean(axis=1) * 10.0
